# baseline (speedup 1.0000x reference)
import functools

import jax
import jax.numpy as jnp
from jax import lax
from jax.experimental import pallas as pl
from jax.experimental.pallas import tpu as pltpu
from jax.experimental.pallas import tpu_sc as plsc

VOCAB = 1_000_000
EMB = 64
BATCH = 4096
SEQ = 200
S2 = SEQ // 2
OUT_W = EMB * S2

_info = plsc.get_sparse_core_info()
NC, NS = _info.num_cores, _info.num_subcores
NW = NC * NS
B_PER_W = BATCH // NW


def _fused_embed(ids, weight):
    mesh = plsc.VectorSubcoreMesh(core_axis_name="c", subcore_axis_name="s")

    @functools.partial(
        pl.kernel,
        mesh=mesh,
        out_type=jax.ShapeDtypeStruct((BATCH, OUT_W), jnp.int32),
        scratch_types=[
            pltpu.VMEM((B_PER_W, SEQ), jnp.int32),
            pltpu.VMEM((SEQ, EMB // 2), jnp.int32),
            pltpu.VMEM((OUT_W,), jnp.int32),
            pltpu.SemaphoreType.DMA,
            pltpu.SemaphoreType.DMA,
        ],
        compiler_params=pltpu.CompilerParams(
            use_tc_tiling_on_sc=False, needs_layout_passes=False
        ),
    )
    def k(ids_hbm, w_hbm, out_hbm, ids_v, g_v, t_v, sem_g, sem_w):
        wid = lax.axis_index("s") * NC + lax.axis_index("c")
        b0 = wid * B_PER_W
        pltpu.sync_copy(ids_hbm.at[pl.ds(b0, B_PER_W)], ids_v)

        lane = lax.broadcasted_iota(jnp.int32, (16,), 0)
        p_even = lane * (2 * S2)
        lo_mask = jnp.full((16,), 0xFFFF, jnp.int32)
        hi_mask = jnp.full((16,), -0x10000, jnp.int32)

        def body(bl, _):
            cp0 = pltpu.async_copy(
                w_hbm.at[ids_v.at[bl, pl.ds(0, 128)]],
                g_v.at[pl.ds(0, 128)],
                sem_g,
            )
            cp1 = pltpu.async_copy(
                w_hbm.at[ids_v.at[bl, pl.ds(128, 72)]],
                g_v.at[pl.ds(128, 72)],
                sem_g,
            )
            cp0.wait()
            cp1.wait()

            def col(s2, _):
                a = g_v[2 * s2, pl.ds(0, 16)]
                b = g_v[2 * s2 + 1, pl.ds(0, 16)]
                a2 = g_v[2 * s2, pl.ds(16, 16)]
                b2 = g_v[2 * s2 + 1, pl.ds(16, 16)]
                c0 = (a & lo_mask) | lax.shift_left(b, 16)
                c1 = lax.shift_right_logical(a, 16) | (b & hi_mask)
                c2 = (a2 & lo_mask) | lax.shift_left(b2, 16)
                c3 = lax.shift_right_logical(a2, 16) | (b2 & hi_mask)
                plsc.store_scatter(t_v, [p_even + s2], c0)
                plsc.store_scatter(t_v, [p_even + (S2 + s2)], c1)
                plsc.store_scatter(t_v, [p_even + (32 * S2 + s2)], c2)
                plsc.store_scatter(t_v, [p_even + (33 * S2 + s2)], c3)
                return ()

            lax.fori_loop(0, S2, col, ())
            pltpu.async_copy(t_v, out_hbm.at[b0 + bl], sem_w).wait()
            return ()

        lax.fori_loop(0, B_PER_W, body, ())

    return k(ids, weight)


def kernel(ids, weight):
    w_i32 = lax.bitcast_convert_type(
        weight.reshape(VOCAB, EMB // 2, 2), jnp.int32
    )
    out_i32 = _fused_embed(ids, w_i32)
    out = lax.bitcast_convert_type(out_i32, jnp.float16)
    return out.reshape(BATCH, EMB, SEQ)


# speedup vs baseline: 1.3202x; 1.3202x over previous
import functools

import jax
import jax.numpy as jnp
from jax import lax
from jax.experimental import pallas as pl
from jax.experimental.pallas import tpu as pltpu
from jax.experimental.pallas import tpu_sc as plsc

VOCAB = 1_000_000
EMB = 64
BATCH = 4096
SEQ = 200
S2 = SEQ // 2
OUT_W = EMB * S2

_info = plsc.get_sparse_core_info()
NC, NS = _info.num_cores, _info.num_subcores
NW = NC * NS
B_PER_W = BATCH // NW


def _fused_embed(ids, weight):
    mesh = plsc.VectorSubcoreMesh(core_axis_name="c", subcore_axis_name="s")

    @functools.partial(
        pl.kernel,
        mesh=mesh,
        out_type=jax.ShapeDtypeStruct((BATCH, OUT_W), jnp.int32),
        scratch_types=[
            pltpu.VMEM((B_PER_W, SEQ), jnp.int32),
            pltpu.VMEM((SEQ, 256), jnp.float16),
            pltpu.VMEM((OUT_W,), jnp.int32),
            pltpu.VMEM((SEQ,), jnp.int32),
            pltpu.VMEM((SEQ + 24,), jnp.int32),
            pltpu.SemaphoreType.DMA,
            pltpu.SemaphoreType.DMA,
        ],
        compiler_params=pltpu.CompilerParams(
            use_tc_tiling_on_sc=False, needs_layout_passes=False
        ),
    )
    def k(ids_hbm, w_hbm, out_hbm, ids_v, g_v, t_v, q_v, o_v, sem_g, sem_w):
        wid = lax.axis_index("s") * NC + lax.axis_index("c")
        b0 = wid * B_PER_W
        pltpu.sync_copy(ids_hbm.at[pl.ds(b0, B_PER_W)], ids_v)

        lane = lax.broadcasted_iota(jnp.int32, (16,), 0)
        p_even = lane * (2 * S2)
        lo_mask = jnp.full((16,), 0xFFFF, jnp.int32)
        hi_mask = jnp.full((16,), -0x10000, jnp.int32)

        def body(bl, _):
            for c in range(13):
                st = c * 16 if c < 12 else SEQ - 16
                idv = ids_v[bl, pl.ds(st, 16)]
                q_v[pl.ds(st, 16)] = lax.shift_right_logical(idv, 2)
                o_v[pl.ds(st, 16)] = lax.shift_left(idv & jnp.int32(3), 6)

            cp0 = pltpu.async_copy(
                w_hbm.at[q_v.at[pl.ds(0, 128)]],
                g_v.at[pl.ds(0, 128)],
                sem_g,
            )
            cp1 = pltpu.async_copy(
                w_hbm.at[q_v.at[pl.ds(128, 72)]],
                g_v.at[pl.ds(128, 72)],
                sem_g,
            )
            cp0.wait()
            cp1.wait()

            def col(s2, _):
                ov = o_v[pl.ds(2 * s2, 16)]
                o0 = ov[0]
                o1 = ov[1]
                a = plsc.bitcast(g_v[2 * s2, pl.ds(o0, 32)], jnp.int32)
                b = plsc.bitcast(g_v[2 * s2 + 1, pl.ds(o1, 32)], jnp.int32)
                a2 = plsc.bitcast(g_v[2 * s2, pl.ds(o0 + 32, 32)], jnp.int32)
                b2 = plsc.bitcast(g_v[2 * s2 + 1, pl.ds(o1 + 32, 32)], jnp.int32)
                c0 = (a & lo_mask) | lax.shift_left(b, 16)
                c1 = lax.shift_right_logical(a, 16) | (b & hi_mask)
                c2 = (a2 & lo_mask) | lax.shift_left(b2, 16)
                c3 = lax.shift_right_logical(a2, 16) | (b2 & hi_mask)
                plsc.store_scatter(t_v, [p_even + s2], c0)
                plsc.store_scatter(t_v, [p_even + (S2 + s2)], c1)
                plsc.store_scatter(t_v, [p_even + (32 * S2 + s2)], c2)
                plsc.store_scatter(t_v, [p_even + (33 * S2 + s2)], c3)
                return ()

            lax.fori_loop(0, S2, col, ())
            pltpu.async_copy(t_v, out_hbm.at[b0 + bl], sem_w).wait()
            return ()

        lax.fori_loop(0, B_PER_W, body, ())

    return k(ids, weight)


def kernel(ids, weight):
    w4 = weight.reshape(VOCAB // 4, 256)
    out_i32 = _fused_embed(ids, w4)
    out = lax.bitcast_convert_type(out_i32, jnp.float16)
    return out.reshape(BATCH, EMB, SEQ)


# speedup vs baseline: 1.5491x; 1.1734x over previous
import functools

import jax
import jax.numpy as jnp
from jax import lax
from jax.experimental import pallas as pl
from jax.experimental.pallas import tpu as pltpu
from jax.experimental.pallas import tpu_sc as plsc

VOCAB = 1_000_000
EMB = 64
BATCH = 4096
SEQ = 200
S2 = SEQ // 2
OUT_W = EMB * S2

_info = plsc.get_sparse_core_info()
NC, NS = _info.num_cores, _info.num_subcores
NW = NC * NS
B_PER_W = BATCH // NW


def _fused_embed(ids, weight):
    mesh = plsc.VectorSubcoreMesh(core_axis_name="c", subcore_axis_name="s")

    @functools.partial(
        pl.kernel,
        mesh=mesh,
        out_type=jax.ShapeDtypeStruct((BATCH, OUT_W), jnp.int32),
        scratch_types=[
            pltpu.VMEM((B_PER_W, SEQ), jnp.int32),
            pltpu.VMEM((SEQ, EMB), jnp.float16),
            pltpu.VMEM((OUT_W,), jnp.int32),
            pltpu.SemaphoreType.DMA,
            pltpu.SemaphoreType.DMA,
        ],
        compiler_params=pltpu.CompilerParams(
            use_tc_tiling_on_sc=False, needs_layout_passes=False
        ),
    )
    def k(ids_hbm, w_hbm, out_hbm, ids_v, g_v, t_v, sem_g, sem_w):
        wid = lax.axis_index("s") * NC + lax.axis_index("c")
        b0 = wid * B_PER_W
        pltpu.sync_copy(ids_hbm.at[pl.ds(b0, B_PER_W)], ids_v)

        lane = lax.broadcasted_iota(jnp.int32, (16,), 0)
        p_even = lane * (2 * S2)
        lo_mask = jnp.full((16,), 0xFFFF, jnp.int32)
        hi_mask = jnp.full((16,), -0x10000, jnp.int32)

        def body(bl, _):
            cp0 = pltpu.async_copy(
                w_hbm.at[ids_v.at[bl, pl.ds(0, 128)]],
                g_v.at[pl.ds(0, 128)],
                sem_g,
            )
            cp1 = pltpu.async_copy(
                w_hbm.at[ids_v.at[bl, pl.ds(128, 72)]],
                g_v.at[pl.ds(128, 72)],
                sem_g,
            )
            cp0.wait()
            cp1.wait()

            def col(s2, _):
                a = plsc.bitcast(g_v[2 * s2, pl.ds(0, 32)], jnp.int32)
                b = plsc.bitcast(g_v[2 * s2 + 1, pl.ds(0, 32)], jnp.int32)
                a2 = plsc.bitcast(g_v[2 * s2, pl.ds(32, 32)], jnp.int32)
                b2 = plsc.bitcast(g_v[2 * s2 + 1, pl.ds(32, 32)], jnp.int32)
                c0 = (a & lo_mask) | lax.shift_left(b, 16)
                c1 = lax.shift_right_logical(a, 16) | (b & hi_mask)
                c2 = (a2 & lo_mask) | lax.shift_left(b2, 16)
                c3 = lax.shift_right_logical(a2, 16) | (b2 & hi_mask)
                plsc.store_scatter(t_v, [p_even + s2], c0)
                plsc.store_scatter(t_v, [p_even + (S2 + s2)], c1)
                plsc.store_scatter(t_v, [p_even + (32 * S2 + s2)], c2)
                plsc.store_scatter(t_v, [p_even + (33 * S2 + s2)], c3)
                return ()

            lax.fori_loop(0, S2, col, ())
            pltpu.async_copy(t_v, out_hbm.at[b0 + bl], sem_w).wait()
            return ()

        lax.fori_loop(0, B_PER_W, body, ())

    return k(ids, weight)


def kernel(ids, weight):
    out_i32 = _fused_embed(ids, weight)
    out = lax.bitcast_convert_type(out_i32, jnp.float16)
    return out.reshape(BATCH, EMB, SEQ)
